# baseline (speedup 1.0000x reference)
import functools

import jax
import jax.numpy as jnp
from jax import lax
from jax.experimental import pallas as pl
from jax.experimental.pallas import tpu as pltpu
from jax.experimental.pallas import tpu_sc as plsc

EMB_DIM = 64
NUM_CORES = 2
NUM_SUBCORES = 16
NUM_WORKERS = NUM_CORES * NUM_SUBCORES
CHUNK = 1024


def _gather_kernel(n_rows):
    b_per_w = n_rows // NUM_WORKERS
    n_chunks = b_per_w // CHUNK
    mesh = plsc.VectorSubcoreMesh(core_axis_name="c", subcore_axis_name="s")

    @functools.partial(
        pl.kernel,
        mesh=mesh,
        out_type=jax.ShapeDtypeStruct((n_rows, EMB_DIM), jnp.float32),
        scratch_types=[
            pltpu.VMEM((CHUNK,), jnp.int32),
            pltpu.VMEM((CHUNK, EMB_DIM), jnp.float32),
            pltpu.SemaphoreType.DMA,
        ],
        compiler_params=pltpu.CompilerParams(use_tc_tiling_on_sc=False),
    )
    def k(idx_hbm, table_hbm, out_hbm, idx_v, rows_v, sem):
        wid = lax.axis_index("s") * NUM_CORES + lax.axis_index("c")
        base = wid * b_per_w

        def body(i, carry):
            off = base + i * CHUNK
            pltpu.sync_copy(idx_hbm.at[pl.ds(off, CHUNK)], idx_v)
            pltpu.async_copy(table_hbm.at[idx_v], rows_v, sem).wait()
            pltpu.sync_copy(rows_v, out_hbm.at[pl.ds(off, CHUNK)])
            return carry

        lax.fori_loop(0, n_chunks, body, 0)

    return k


def kernel(indices, weight):
    shape = indices.shape
    idx_flat = indices.reshape(-1)
    out = _gather_kernel(idx_flat.shape[0])(idx_flat, weight)
    return out.reshape(*shape, EMB_DIM)


# speedup vs baseline: 1.0139x; 1.0139x over previous
import functools

import jax
import jax.numpy as jnp
from jax import lax
from jax.experimental import pallas as pl
from jax.experimental.pallas import tpu as pltpu
from jax.experimental.pallas import tpu_sc as plsc

EMB_DIM = 64
NUM_CORES = 2
NUM_SUBCORES = 16
NUM_WORKERS = NUM_CORES * NUM_SUBCORES
CHUNK = 512
NBUF = 3


def _gather_kernel(n_rows):
    b_per_w = n_rows // NUM_WORKERS
    n_chunks = b_per_w // CHUNK
    assert n_chunks * CHUNK == b_per_w and n_chunks > NBUF + 2
    tail = 2 + (n_chunks - 2) % NBUF
    n_main = n_chunks - tail
    assert n_main % NBUF == 0 and n_main // NBUF >= 1
    mesh = plsc.VectorSubcoreMesh(core_axis_name="c", subcore_axis_name="s")

    @functools.partial(
        pl.kernel,
        mesh=mesh,
        out_type=jax.ShapeDtypeStruct((n_rows, EMB_DIM), jnp.float32),
        scratch_types=[
            pltpu.VMEM((b_per_w,), jnp.int32),
            [pltpu.VMEM((CHUNK, EMB_DIM), jnp.float32) for _ in range(NBUF)],
            [pltpu.SemaphoreType.DMA for _ in range(NBUF)],
            [pltpu.SemaphoreType.DMA for _ in range(NBUF)],
        ],
        compiler_params=pltpu.CompilerParams(use_tc_tiling_on_sc=False),
    )
    def k(idx_hbm, table_hbm, out_hbm, idx_v, rows, gsem, osem):
        wid = lax.axis_index("s") * NUM_CORES + lax.axis_index("c")
        base = wid * b_per_w

        pltpu.sync_copy(idx_hbm.at[pl.ds(base, b_per_w)], idx_v)

        def gather_desc(i, b):
            return pltpu.make_async_copy(
                table_hbm.at[idx_v.at[pl.ds(i * CHUNK, CHUNK)]],
                rows[b], gsem[b])

        def put_desc(i, b):
            return pltpu.make_async_copy(
                rows[b], out_hbm.at[pl.ds(base + i * CHUNK, CHUNK)], osem[b])

        def step(i, b, skip_put_wait=False):
            gather_desc(i, b).wait()
            put_desc(i, b).start()
            if not skip_put_wait:
                put_desc(i - 1, (b - 1) % NBUF).wait()
            gather_desc(i + 2, (b + 2) % NBUF).start()

        gather_desc(0, 0).start()
        gather_desc(1, 1).start()
        step(0, 0, skip_put_wait=True)
        for u in range(1, NBUF):
            step(u, u)

        def body(r, carry):
            i = r * NBUF
            for u in range(NBUF):
                step(i + u, u)
            return carry

        lax.fori_loop(1, n_main // NBUF, body, 0)

        for i in range(n_main, n_chunks):
            b = i % NBUF
            gather_desc(i, b).wait()
            put_desc(i, b).start()
            if i + 2 < n_chunks:
                put_desc(i - 1, (b - 1) % NBUF).wait()
                gather_desc(i + 2, (i + 2) % NBUF).start()
        for i in range(n_chunks - NBUF, n_chunks):
            put_desc(i, i % NBUF).wait()

    return k


def kernel(indices, weight):
    shape = indices.shape
    idx_flat = indices.reshape(-1)
    out = _gather_kernel(idx_flat.shape[0])(idx_flat, weight)
    return out.reshape(*shape, EMB_DIM)
